# baseline (speedup 1.0000x reference)
import functools

import jax
import jax.numpy as jnp
from jax import lax
from jax.experimental import pallas as pl
from jax.experimental.pallas import tpu as pltpu
from jax.experimental.pallas import tpu_sc as plsc

D_MODEL = 128
SEQ = 64
BATCH_BLOCK = 256

_INFO = plsc.get_sparse_core_info()
_NC = _INFO.num_cores
_NWORK = 8
_ROWS_PER_W = SEQ // _NWORK


def _sc_gather(pe64, inds):
    mesh = plsc.VectorSubcoreMesh(core_axis_name="c", subcore_axis_name="s")

    @functools.partial(
        pl.kernel,
        mesh=mesh,
        out_type=jax.ShapeDtypeStruct((SEQ, D_MODEL), jnp.float32),
        scratch_types=[
            pltpu.VMEM((_ROWS_PER_W,), jnp.int32),
            pltpu.VMEM((_ROWS_PER_W, D_MODEL), jnp.float32),
            pltpu.SemaphoreType.DMA,
        ],
    )
    def gather_k(table_hbm, idx_hbm, out_hbm, idx_v, rows_v, sem):
        wid = lax.axis_index("s") * _NC + lax.axis_index("c")

        @pl.when(wid < _NWORK)
        def _():
            base = wid * _ROWS_PER_W
            pltpu.sync_copy(idx_hbm.at[pl.ds(base, _ROWS_PER_W)], idx_v)
            pltpu.async_copy(table_hbm.at[idx_v], rows_v, sem).wait()
            pltpu.sync_copy(rows_v, out_hbm.at[pl.ds(base, _ROWS_PER_W)])

    return gather_k(pe64, inds)


def _add_body(x_ref, fp_ref, o_ref):
    o_ref[...] = x_ref[...] + fp_ref[...][None, :, :]


def _tc_add(x, fp):
    nb = x.shape[0] // BATCH_BLOCK
    return pl.pallas_call(
        _add_body,
        grid=(nb,),
        in_specs=[
            pl.BlockSpec((BATCH_BLOCK, SEQ, D_MODEL), lambda i: (i, 0, 0)),
            pl.BlockSpec((SEQ, D_MODEL), lambda i: (0, 0)),
        ],
        out_specs=pl.BlockSpec((BATCH_BLOCK, SEQ, D_MODEL), lambda i: (i, 0, 0)),
        out_shape=jax.ShapeDtypeStruct(x.shape, x.dtype),
        compiler_params=pltpu.CompilerParams(
            dimension_semantics=("arbitrary",),
        ),
    )(x, fp)


def kernel(x, x_node_inds, pe):
    inds = x_node_inds.astype(jnp.int32)
    pe64 = pe[:SEQ]
    fp = _sc_gather(pe64, inds)
    return _tc_add(x, fp)
